# baseline (speedup 1.0000x reference)
import functools

import jax
import jax.numpy as jnp
from jax import lax
from jax.experimental import pallas as pl
from jax.experimental.pallas import tpu as pltpu
from jax.experimental.pallas import tpu_sc as plsc

_NC = 2
_NS = 16
_NW = _NC * _NS
_L = 16

_NI = 16384
_NJ = 200
_D = 4
_Q = 4
_KI = _NI // _Q
_KO = _KI * _D
_UNITS_PER_TILE = _NJ * _Q // _NW


def _make_sc_gather():
    mesh = plsc.VectorSubcoreMesh(
        core_axis_name="c", subcore_axis_name="s", num_cores=_NC,
        num_subcores=_NS)

    @functools.partial(
        pl.kernel,
        out_type=jax.ShapeDtypeStruct((_NI * _NJ * _D,), jnp.float32),
        mesh=mesh,
        compiler_params=pltpu.CompilerParams(needs_layout_passes=False),
        scratch_types=(
            [pltpu.VMEM((_KI,), jnp.int32) for _ in range(4)]
            + [pltpu.VMEM((_KO,), jnp.float32) for _ in range(3)]
            + [
                pltpu.VMEM((32,), jnp.float32),
                pltpu.SemaphoreType.DMA,
                pltpu.SemaphoreType.DMA,
            ]
        ),
    )
    def sc_gather(xt_hbm, wc_hbm, out_hbm, idx0_v, idx1_v, idx2_v, idx3_v,
                  out0_v, out1_v, out2_v, w_v, in_sem, out_sem):
        idx_bufs = [idx0_v, idx1_v, idx2_v, idx3_v]
        out_bufs = [out0_v, out1_v, out2_v]
        wid = lax.axis_index("s") * _NC + lax.axis_index("c")
        u0 = wid * _UNITS_PER_TILE
        pltpu.sync_copy(wc_hbm, w_v)

        def in_slice(n):
            u = u0 + n
            j = u // _Q
            q = u % _Q
            return xt_hbm.at[j, pl.ds(q * _KI, _KI)]

        def out_slice(n):
            u = u0 + n
            j = u // _Q
            q = u % _Q
            return out_hbm.at[pl.ds(j * (_KO * _Q) + q * _KO, _KO)]

        def compute(ibuf, obuf):
            def inner(m, _):
                for gg in range(8):
                    idx = idx_bufs[ibuf][pl.ds(m * 128 + gg * _L, _L)]
                    for dd in range(_D):
                        vals = plsc.load_gather(w_v, [idx + dd * 8])
                        out_bufs[obuf][
                            pl.ds(m * 512 + dd * 128 + gg * _L, _L)] = vals
                return 0
            lax.fori_loop(0, _KI // 128, inner, 0)

        n_pre = min(3, _UNITS_PER_TILE)
        in_copies = [
            pltpu.async_copy(in_slice(n), idx_bufs[n % 4], in_sem)
            for n in range(n_pre)]
        out_copies = [None, None, None]
        for n in range(_UNITS_PER_TILE):
            ibuf, obuf = n % 4, n % 3
            in_copies.pop(0).wait()
            if n + n_pre < _UNITS_PER_TILE:
                in_copies.append(pltpu.async_copy(
                    in_slice(n + n_pre), idx_bufs[(n + n_pre) % 4], in_sem))
            if out_copies[obuf] is not None:
                out_copies[obuf].wait()
            compute(ibuf, obuf)
            out_copies[obuf] = pltpu.async_copy(
                out_bufs[obuf], out_slice(n), out_sem)

        for cp in out_copies:
            if cp is not None:
                cp.wait()

    return sc_gather


@functools.lru_cache(maxsize=None)
def _sc_gather_fn():
    return _make_sc_gather()


@jax.jit
def kernel(x, weight):
    wc = jnp.pad(weight, ((0, 8 - weight.shape[0]), (0, 0))).T.reshape(-1)
    f = _sc_gather_fn()(x.T, wc)
    return (f.reshape(_NJ, _NI // 128, _D, 128)
            .transpose(1, 3, 0, 2)
            .reshape(_NI, _NJ, _D))
